# baseline (speedup 1.0000x reference)
import jax
import jax.numpy as jnp
from jax.experimental import pallas as pl
from jax.experimental.pallas import tpu as pltpu

N_TOK = 4096
D_MODEL = 1024
NUM_SPEC = 8
NUM_SHARED = 2
NUM_TOTAL = NUM_SPEC + NUM_SHARED
BT = 2048


def _routing(slT, shlT):
    m = jnp.max(slT, axis=0, keepdims=True)
    ex = jnp.exp(slT - m)
    z = jnp.sum(ex, axis=0, keepdims=True)
    sub8 = jax.lax.broadcasted_iota(jnp.int32, slT.shape, 0)
    g1 = jnp.max(ex, axis=0, keepdims=True)
    a1 = jnp.min(jnp.where(ex == g1, sub8, NUM_SPEC), axis=0, keepdims=True)
    ex2 = jnp.where(sub8 == a1, 0.0, ex)
    g2 = jnp.max(ex2, axis=0, keepdims=True)
    a2 = jnp.min(jnp.where(ex2 == g2, sub8, NUM_SPEC), axis=0, keepdims=True)
    denom = g1 + g2 + 1e-6 * z
    w1 = g1 / denom
    w2 = g2 / denom
    sub16 = jax.lax.broadcasted_iota(jnp.int32, (16, slT.shape[1]), 0)
    ct = w1 * (sub16 == a1).astype(jnp.float32) + w2 * (sub16 == a2).astype(jnp.float32)
    sm = jnp.max(shlT, axis=0, keepdims=True)
    sex = jnp.exp(shlT - sm)
    ssum = jnp.sum(sex, axis=0, keepdims=True)
    ct = ct + (sex[0:1, :] / ssum) * (sub16 == NUM_SPEC).astype(jnp.float32)
    ct = ct + (sex[1:2, :] / ssum) * (sub16 == NUM_SPEC + 1).astype(jnp.float32)
    return ct


def _moe_body(slT_ref, shlT_ref, x_ref, wspec_ref, wshared_ref, o_ref, ct_ref):
    p = pl.program_id(0)
    t = pl.program_id(1)

    @pl.when((t == 0) & (p == 0))
    def _do_routing():
        ct_ref[...] = _routing(slT_ref[...], shlT_ref[...]).astype(jnp.bfloat16)

    lane256 = jax.lax.broadcasted_iota(jnp.int32, (16, 256), 1)
    sub16 = jax.lax.broadcasted_iota(jnp.int32, (16, 256), 0)
    target = jnp.where(lane256 < 128, 2 * p, 2 * p + 1)
    onehot2 = (sub16 == target).astype(jnp.bfloat16)
    scale_nar = jax.lax.dot_general(
        ct_ref[:, pl.ds(t * BT, BT)], onehot2, (((0,), (0,)), ((), ())),
        preferred_element_type=jnp.float32)
    s16 = scale_nar.astype(jnp.bfloat16)
    rep = D_MODEL // 128
    bcA = jnp.concatenate([s16[:, 0:128]] * rep, axis=1)
    bcB = jnp.concatenate([s16[:, 128:256]] * rep, axis=1)
    x_blk = x_ref[...]
    xs2 = jnp.concatenate([bcA * x_blk, bcB * x_blk], axis=1)

    def _acc(w_ref):
        contrib = jnp.dot(xs2, w_ref[0], preferred_element_type=jnp.float32)

        @pl.when(p == 0)
        def _init():
            o_ref[pl.ds(t * BT, BT), :] = contrib

        @pl.when(p > 0)
        def _add():
            o_ref[pl.ds(t * BT, BT), :] += contrib

    @pl.when(p < NUM_SPEC // 2)
    def _spec():
        _acc(wspec_ref)

    @pl.when(p >= NUM_SPEC // 2)
    def _shared():
        _acc(wshared_ref)


def kernel(x, spec_router_logits, shared_router_logits, spec_expert_weights, shared_expert_weights):
    x16 = x.astype(jnp.bfloat16)
    wspec4 = spec_expert_weights.reshape(NUM_SPEC // 2, 2 * D_MODEL, D_MODEL).astype(jnp.bfloat16)
    wshared1 = shared_expert_weights.reshape(NUM_SHARED // 2, 2 * D_MODEL, D_MODEL).astype(jnp.bfloat16)
    npair = NUM_TOTAL // 2
    return pl.pallas_call(
        _moe_body,
        grid=(npair, N_TOK // BT),
        in_specs=[
            pl.BlockSpec((NUM_SPEC, N_TOK), lambda p, t: (0, 0)),
            pl.BlockSpec((NUM_SHARED, N_TOK), lambda p, t: (0, 0)),
            pl.BlockSpec((BT, D_MODEL), lambda p, t: (t, 0)),
            pl.BlockSpec((1, 2 * D_MODEL, D_MODEL), lambda p, t: (jnp.minimum(p, NUM_SPEC // 2 - 1), 0, 0)),
            pl.BlockSpec((1, 2 * D_MODEL, D_MODEL), lambda p, t: (0, 0, 0)),
        ],
        out_specs=pl.BlockSpec((N_TOK, D_MODEL), lambda p, t: (0, 0)),
        out_shape=jax.ShapeDtypeStruct((N_TOK, D_MODEL), jnp.float32),
        scratch_shapes=[pltpu.VMEM((16, N_TOK), jnp.bfloat16)],
        compiler_params=pltpu.CompilerParams(
            dimension_semantics=("arbitrary", "arbitrary"),
        ),
    )(spec_router_logits.T, shared_router_logits.T, x16, wspec4, wshared1)


# speedup vs baseline: 1.1849x; 1.1849x over previous
import jax
import jax.numpy as jnp
from jax.experimental import pallas as pl
from jax.experimental.pallas import tpu as pltpu

N_TOK = 4096
D_MODEL = 1024
NUM_SPEC = 8
NUM_SHARED = 2
NUM_TOTAL = NUM_SPEC + NUM_SHARED
BT = 1024


def _routing(slT, shlT):
    m = jnp.max(slT, axis=0, keepdims=True)
    ex = jnp.exp(slT - m)
    z = jnp.sum(ex, axis=0, keepdims=True)
    sub8 = jax.lax.broadcasted_iota(jnp.int32, slT.shape, 0)
    g1 = jnp.max(ex, axis=0, keepdims=True)
    a1 = jnp.min(jnp.where(ex == g1, sub8, NUM_SPEC), axis=0, keepdims=True)
    ex2 = jnp.where(sub8 == a1, 0.0, ex)
    g2 = jnp.max(ex2, axis=0, keepdims=True)
    a2 = jnp.min(jnp.where(ex2 == g2, sub8, NUM_SPEC), axis=0, keepdims=True)
    denom = g1 + g2 + 1e-6 * z
    w1 = g1 / denom
    w2 = g2 / denom
    sub16 = jax.lax.broadcasted_iota(jnp.int32, (16, slT.shape[1]), 0)
    ct = w1 * (sub16 == a1).astype(jnp.float32) + w2 * (sub16 == a2).astype(jnp.float32)
    sm = jnp.max(shlT, axis=0, keepdims=True)
    sex = jnp.exp(shlT - sm)
    ssum = jnp.sum(sex, axis=0, keepdims=True)
    ct = ct + (sex[0:1, :] / ssum) * (sub16 == NUM_SPEC).astype(jnp.float32)
    ct = ct + (sex[1:2, :] / ssum) * (sub16 == NUM_SPEC + 1).astype(jnp.float32)
    return ct


def _moe_body(slT_ref, shlT_ref, x_ref, wspec_ref, wshared_ref, o_ref, ct_ref, wb_ref):
    p = pl.program_id(0)
    ts = pl.program_id(1)
    t = jnp.maximum(ts - 1, 0)

    @pl.when(ts == 0)
    def _prep():
        @pl.when(p == 0)
        def _do_routing():
            ct_ref[...] = _routing(slT_ref[...], shlT_ref[...]).astype(jnp.bfloat16)

        @pl.when(p < NUM_SPEC // 2)
        def _s():
            wb_ref[...] = wspec_ref[0].astype(jnp.bfloat16)

        @pl.when(p >= NUM_SPEC // 2)
        def _h():
            wb_ref[...] = wshared_ref[0].astype(jnp.bfloat16)

    @pl.when(ts > 0)
    def _compute():
        _compute_tile(p, t, x_ref, o_ref, ct_ref, wb_ref)


def _compute_tile(p, t, x_ref, o_ref, ct_ref, wb_ref):

    lane256 = jax.lax.broadcasted_iota(jnp.int32, (16, 256), 1)
    sub16 = jax.lax.broadcasted_iota(jnp.int32, (16, 256), 0)
    target = jnp.where(lane256 < 128, 2 * p, 2 * p + 1)
    onehot2 = (sub16 == target).astype(jnp.bfloat16)
    scale_nar = jax.lax.dot_general(
        ct_ref[:, pl.ds(t * BT, BT)], onehot2, (((0,), (0,)), ((), ())),
        preferred_element_type=jnp.float32)
    s16 = scale_nar.astype(jnp.bfloat16)
    rep = D_MODEL // 128
    bcA = jnp.concatenate([s16[:, 0:128]] * rep, axis=1)
    bcB = jnp.concatenate([s16[:, 128:256]] * rep, axis=1)
    x_blk = x_ref[...]
    xs2 = jnp.concatenate([bcA * x_blk, bcB * x_blk], axis=1)

    contrib = jnp.dot(xs2, wb_ref[...], preferred_element_type=jnp.float32)

    @pl.when(p == 0)
    def _init():
        o_ref[pl.ds(t * BT, BT), :] = contrib

    @pl.when(p > 0)
    def _add():
        o_ref[pl.ds(t * BT, BT), :] += contrib


def kernel(x, spec_router_logits, shared_router_logits, spec_expert_weights, shared_expert_weights):
    x16 = x.astype(jnp.bfloat16)
    wspec4 = spec_expert_weights.reshape(NUM_SPEC // 2, 2 * D_MODEL, D_MODEL)
    wshared1 = shared_expert_weights.reshape(NUM_SHARED // 2, 2 * D_MODEL, D_MODEL)
    npair = NUM_TOTAL // 2
    return pl.pallas_call(
        _moe_body,
        grid=(npair, N_TOK // BT + 1),
        in_specs=[
            pl.BlockSpec((NUM_SPEC, N_TOK), lambda p, t: (0, 0)),
            pl.BlockSpec((NUM_SHARED, N_TOK), lambda p, t: (0, 0)),
            pl.BlockSpec((BT, D_MODEL), lambda p, t: (jnp.maximum(t - 1, 0), 0)),
            pl.BlockSpec((1, 2 * D_MODEL, D_MODEL), lambda p, t: (jnp.minimum(p, NUM_SPEC // 2 - 1), 0, 0)),
            pl.BlockSpec((1, 2 * D_MODEL, D_MODEL), lambda p, t: (0, 0, 0)),
        ],
        out_specs=pl.BlockSpec((N_TOK, D_MODEL), lambda p, t: (0, 0)),
        out_shape=jax.ShapeDtypeStruct((N_TOK, D_MODEL), jnp.float32),
        scratch_shapes=[pltpu.VMEM((16, N_TOK), jnp.bfloat16),
                        pltpu.VMEM((2 * D_MODEL, D_MODEL), jnp.bfloat16)],
        compiler_params=pltpu.CompilerParams(
            dimension_semantics=("arbitrary", "arbitrary"),
        ),
    )(spec_router_logits.T, shared_router_logits.T, x16, wspec4, wshared1)
